# baseline (speedup 1.0000x reference)
import jax
import jax.numpy as jnp
from jax.experimental import pallas as pl
from jax.experimental.pallas import tpu as pltpu

B, J, H, W = 64, 17, 64, 48
HW = H * W
TOPK_K = 8
BB = 8


def _ohkm_kernel(w_ref, p_ref, t_ref, out_ref, loss_ref):
    i = pl.program_id(0)
    p = p_ref[...]
    t = t_ref[...]
    w = w_ref[...]
    diff = (p - t) * w[..., None]
    s = jnp.sum(diff * diff, axis=2)
    loss_ref[pl.ds(i * BB, BB), :] = s * (0.5 / HW)

    @pl.when(i == pl.num_programs(0) - 1)
    def _finalize():
        v = loss_ref[...]
        col = jax.lax.broadcasted_iota(jnp.int32, (B, J), 1)
        acc = jnp.zeros((B,), jnp.float32)
        for _ in range(TOPK_K):
            m = jnp.max(v, axis=1)
            eq = v == m[:, None]
            idx = jnp.min(jnp.where(eq, col, J), axis=1)
            acc = acc + m
            v = jnp.where(col == idx[:, None], -jnp.inf, v)
        out_ref[0, 0] = jnp.sum(acc) * (1.0 / (TOPK_K * B))


def kernel(pred, target, target_weight):
    p = pred.reshape(B, J, HW)
    t = target.reshape(B, J, HW)
    w = target_weight.reshape(B, J)
    out = pl.pallas_call(
        _ohkm_kernel,
        grid=(B // BB,),
        in_specs=[
            pl.BlockSpec((BB, J), lambda i: (i, 0)),
            pl.BlockSpec((BB, J, HW), lambda i: (i, 0, 0)),
            pl.BlockSpec((BB, J, HW), lambda i: (i, 0, 0)),
        ],
        out_specs=pl.BlockSpec((1, 1), lambda i: (0, 0), memory_space=pltpu.SMEM),
        out_shape=jax.ShapeDtypeStruct((1, 1), jnp.float32),
        scratch_shapes=[pltpu.VMEM((B, J), jnp.float32)],
    )(w, p, t)
    return out[0, 0]


# speedup vs baseline: 1.5074x; 1.5074x over previous
import jax
import jax.numpy as jnp
from jax.experimental import pallas as pl
from jax.experimental.pallas import tpu as pltpu

B, J, H, W = 64, 17, 64, 48
HW = H * W
TOPK_K = 8
BB = 8


def _ohkm_kernel(w_ref, p_ref, t_ref, out_ref, loss_ref):
    i = pl.program_id(0)
    p = p_ref[...]
    t = t_ref[...]
    w = w_ref[...]
    diff = (p - t) * w[..., None]
    s = jnp.sum(diff * diff, axis=(2, 3))
    loss_ref[pl.ds(i * BB, BB), :] = s * (0.5 / HW)

    @pl.when(i == pl.num_programs(0) - 1)
    def _finalize():
        v = loss_ref[...]
        col = jax.lax.broadcasted_iota(jnp.int32, (B, J), 1)
        acc = jnp.zeros((B,), jnp.float32)
        for _ in range(TOPK_K):
            m = jnp.max(v, axis=1)
            eq = v == m[:, None]
            idx = jnp.min(jnp.where(eq, col, J), axis=1)
            acc = acc + m
            v = jnp.where(col == idx[:, None], -jnp.inf, v)
        out_ref[0, 0] = jnp.sum(acc) * (1.0 / (TOPK_K * B))


def kernel(pred, target, target_weight):
    out = pl.pallas_call(
        _ohkm_kernel,
        grid=(B // BB,),
        in_specs=[
            pl.BlockSpec((BB, J, 1), lambda i: (i, 0, 0)),
            pl.BlockSpec((BB, J, H, W), lambda i: (i, 0, 0, 0)),
            pl.BlockSpec((BB, J, H, W), lambda i: (i, 0, 0, 0)),
        ],
        out_specs=pl.BlockSpec((1, 1), lambda i: (0, 0), memory_space=pltpu.SMEM),
        out_shape=jax.ShapeDtypeStruct((1, 1), jnp.float32),
        scratch_shapes=[pltpu.VMEM((B, J), jnp.float32)],
    )(target_weight, pred, target)
    return out[0, 0]
